# initial kernel scaffold (unmeasured)
import jax
import jax.numpy as jnp
from jax import lax
from jax.experimental import pallas as pl
from jax.experimental.pallas import tpu as pltpu

_DeviceIdType = getattr(pl, "DeviceIdType", None) or pltpu.DeviceIdType
_MESH = _DeviceIdType.MESH
_CompilerParams = getattr(pltpu, "CompilerParams", None) or pltpu.TPUCompilerParams


def kernel(x, dy, gamma):
    m, d = x.shape
    half = m // 2

    def body(x_hbm, dy_hbm, out_ref, x_vmem, dy_vmem, send_ref, comb_ref,
             recv1_ref, recv2_ref, load_sems, send_sems, recv_sems):
        my_x = lax.axis_index("x")
        my_y = lax.axis_index("y")

        barrier = pltpu.get_barrier_semaphore()
        pl.semaphore_signal(barrier, inc=1, device_id=(1 - my_x, my_y),
                            device_id_type=_MESH)
        pl.semaphore_signal(barrier, inc=1, device_id=(my_x, 1 - my_y),
                            device_id_type=_MESH)

        row0 = my_x * half
        cp_x = pltpu.make_async_copy(
            x_hbm.at[pl.ds(row0, half), :], x_vmem, load_sems.at[0])
        cp_dy = pltpu.make_async_copy(
            dy_hbm.at[pl.ds(row0, half), :], dy_vmem, load_sems.at[1])
        cp_x.start()
        cp_dy.start()
        cp_x.wait()
        cp_dy.wait()

        xv = x_vmem[:, :]
        dyv = dy_vmem[:, :]
        mu = jnp.mean(xv, axis=1, keepdims=True)
        xc = xv - mu
        var = jnp.mean(xc * xc, axis=1, keepdims=True)
        xhat = xc * lax.rsqrt(var + 1e-5)
        dgamma = jnp.sum(dyv * xhat, axis=0, keepdims=True)
        dbeta = jnp.sum(dyv, axis=0, keepdims=True)
        send_ref[:, :] = jnp.concatenate([dgamma, dbeta], axis=0)

        pl.semaphore_wait(barrier, 2)

        rdma1 = pltpu.make_async_remote_copy(
            src_ref=send_ref, dst_ref=recv1_ref,
            send_sem=send_sems.at[0], recv_sem=recv_sems.at[0],
            device_id=(1 - my_x, my_y), device_id_type=_MESH)
        rdma1.start()
        rdma1.wait()
        comb_ref[:, :] = send_ref[:, :] + recv1_ref[:, :]

        rdma2 = pltpu.make_async_remote_copy(
            src_ref=comb_ref, dst_ref=recv2_ref,
            send_sem=send_sems.at[1], recv_sem=recv_sems.at[1],
            device_id=(my_x, 1 - my_y), device_id_type=_MESH)
        rdma2.start()
        rdma2.wait()
        out_ref[:, :] = comb_ref[:, :] + recv2_ref[:, :]

    return pl.pallas_call(
        body,
        out_shape=jax.ShapeDtypeStruct((2, d), jnp.float32),
        in_specs=[pl.BlockSpec(memory_space=pltpu.ANY),
                  pl.BlockSpec(memory_space=pltpu.ANY)],
        out_specs=pl.BlockSpec(memory_space=pltpu.VMEM),
        scratch_shapes=[
            pltpu.VMEM((half, d), jnp.float32),
            pltpu.VMEM((half, d), jnp.float32),
            pltpu.VMEM((2, d), jnp.float32),
            pltpu.VMEM((2, d), jnp.float32),
            pltpu.VMEM((2, d), jnp.float32),
            pltpu.VMEM((2, d), jnp.float32),
            pltpu.SemaphoreType.DMA((2,)),
            pltpu.SemaphoreType.DMA((2,)),
            pltpu.SemaphoreType.DMA((2,)),
        ],
        compiler_params=_CompilerParams(collective_id=0),
    )(x, dy)


# baseline (device time: 24086 ns/iter reference)
import jax
import jax.numpy as jnp
from jax import lax
from jax.experimental import pallas as pl
from jax.experimental.pallas import tpu as pltpu

_DeviceIdType = getattr(pl, "DeviceIdType", None) or pltpu.DeviceIdType
_MESH = _DeviceIdType.MESH
_CompilerParams = getattr(pltpu, "CompilerParams", None) or pltpu.TPUCompilerParams


def kernel(x, dy, gamma):
    m, d = x.shape
    half = m // 2

    def body(x_hbm, dy_hbm, out_ref, x_vmem, dy_vmem, send_ref, comb_ref,
             recv1_ref, recv2_ref, load_sems, send_sems, recv_sems):
        my_x = lax.axis_index("x")
        my_y = lax.axis_index("y")

        barrier = pltpu.get_barrier_semaphore()
        pl.semaphore_signal(barrier, inc=1, device_id=(1 - my_x, my_y),
                            device_id_type=_MESH)
        pl.semaphore_signal(barrier, inc=1, device_id=(my_x, 1 - my_y),
                            device_id_type=_MESH)

        row0 = my_x * half
        cp_x = pltpu.make_async_copy(
            x_hbm.at[pl.ds(row0, half), :], x_vmem, load_sems.at[0])
        cp_dy = pltpu.make_async_copy(
            dy_hbm.at[pl.ds(row0, half), :], dy_vmem, load_sems.at[1])
        cp_x.start()
        cp_dy.start()
        cp_x.wait()
        cp_dy.wait()

        xv = x_vmem[:, :]
        dyv = dy_vmem[:, :]
        mu = jnp.mean(xv, axis=1, keepdims=True)
        xc = xv - mu
        var = jnp.mean(xc * xc, axis=1, keepdims=True)
        xhat = xc * lax.rsqrt(var + 1e-5)
        dgamma = jnp.sum(dyv * xhat, axis=0, keepdims=True)
        dbeta = jnp.sum(dyv, axis=0, keepdims=True)
        send_ref[:, :] = jnp.concatenate([dgamma, dbeta], axis=0)

        pl.semaphore_wait(barrier, 2)

        rdma1 = pltpu.make_async_remote_copy(
            src_ref=send_ref, dst_ref=recv1_ref,
            send_sem=send_sems.at[0], recv_sem=recv_sems.at[0],
            device_id=(1 - my_x, my_y), device_id_type=_MESH)
        rdma1.start()
        rdma1.wait()
        comb_ref[:, :] = send_ref[:, :] + recv1_ref[:, :]

        rdma2 = pltpu.make_async_remote_copy(
            src_ref=comb_ref, dst_ref=recv2_ref,
            send_sem=send_sems.at[1], recv_sem=recv_sems.at[1],
            device_id=(my_x, 1 - my_y), device_id_type=_MESH)
        rdma2.start()
        rdma2.wait()
        out_ref[:, :] = comb_ref[:, :] + recv2_ref[:, :]

    return pl.pallas_call(
        body,
        out_shape=jax.ShapeDtypeStruct((2, d), jnp.float32),
        in_specs=[pl.BlockSpec(memory_space=pl.ANY),
                  pl.BlockSpec(memory_space=pl.ANY)],
        out_specs=pl.BlockSpec(memory_space=pltpu.VMEM),
        scratch_shapes=[
            pltpu.VMEM((half, d), jnp.float32),
            pltpu.VMEM((half, d), jnp.float32),
            pltpu.VMEM((2, d), jnp.float32),
            pltpu.VMEM((2, d), jnp.float32),
            pltpu.VMEM((2, d), jnp.float32),
            pltpu.VMEM((2, d), jnp.float32),
            pltpu.SemaphoreType.DMA((2,)),
            pltpu.SemaphoreType.DMA((2,)),
            pltpu.SemaphoreType.DMA((2,)),
        ],
        compiler_params=_CompilerParams(
            collective_id=0, vmem_limit_bytes=100 * 1024 * 1024),
    )(x, dy)


# device time: 20428 ns/iter; 1.1791x vs baseline; 1.1791x over previous
import jax
import jax.numpy as jnp
from jax import lax
from jax.experimental import pallas as pl
from jax.experimental.pallas import tpu as pltpu

_DeviceIdType = getattr(pl, "DeviceIdType", None) or pltpu.DeviceIdType
_MESH = _DeviceIdType.MESH
_CompilerParams = getattr(pltpu, "CompilerParams", None) or pltpu.TPUCompilerParams

_CHUNK = 256


def kernel(x, dy, gamma):
    m, d = x.shape
    half = m // 2
    n_chunks = half // _CHUNK

    def body(x_hbm, dy_hbm, out_ref, x_buf, dy_buf, send_ref, recv_ref,
             load_sems, send_sems, recv_sems):
        my_x = lax.axis_index("x")
        my_y = lax.axis_index("y")
        peers = [(1 - my_x, my_y), (my_x, 1 - my_y), (1 - my_x, 1 - my_y)]

        barrier = pltpu.get_barrier_semaphore()
        for p in peers:
            pl.semaphore_signal(barrier, inc=1, device_id=p,
                                device_id_type=_MESH)

        row0 = my_x * half

        def start_load(h):
            slot = h % 2
            r = row0 + h * _CHUNK
            pltpu.make_async_copy(
                x_hbm.at[pl.ds(r, _CHUNK), :], x_buf.at[slot],
                load_sems.at[slot, 0]).start()
            pltpu.make_async_copy(
                dy_hbm.at[pl.ds(r, _CHUNK), :], dy_buf.at[slot],
                load_sems.at[slot, 1]).start()

        def wait_load(h):
            slot = h % 2
            r = row0 + h * _CHUNK
            pltpu.make_async_copy(
                x_hbm.at[pl.ds(r, _CHUNK), :], x_buf.at[slot],
                load_sems.at[slot, 0]).wait()
            pltpu.make_async_copy(
                dy_hbm.at[pl.ds(r, _CHUNK), :], dy_buf.at[slot],
                load_sems.at[slot, 1]).wait()

        start_load(0)
        dgamma = jnp.zeros((1, d), jnp.float32)
        dbeta = jnp.zeros((1, d), jnp.float32)
        for h in range(n_chunks):
            if h + 1 < n_chunks:
                start_load(h + 1)
            wait_load(h)
            slot = h % 2
            xv = x_buf[slot]
            dyv = dy_buf[slot]
            s1 = jnp.sum(xv, axis=1, keepdims=True)
            s2 = jnp.sum(xv * xv, axis=1, keepdims=True)
            mu = s1 * (1.0 / d)
            var = s2 * (1.0 / d) - mu * mu
            xhat = (xv - mu) * lax.rsqrt(var + 1e-5)
            dgamma = dgamma + jnp.sum(dyv * xhat, axis=0, keepdims=True)
            dbeta = dbeta + jnp.sum(dyv, axis=0, keepdims=True)
        send_ref[:, :] = jnp.concatenate([dgamma, dbeta], axis=0)

        pl.semaphore_wait(barrier, 3)

        rdmas = []
        for k, p in enumerate(peers):
            rdma = pltpu.make_async_remote_copy(
                src_ref=send_ref, dst_ref=recv_ref.at[k],
                send_sem=send_sems.at[k], recv_sem=recv_sems.at[k],
                device_id=p, device_id_type=_MESH)
            rdma.start()
            rdmas.append(rdma)
        for rdma in rdmas:
            rdma.wait()

        out_ref[:, :] = (send_ref[:, :] + recv_ref[0] + recv_ref[1]
                         + recv_ref[2])

    return pl.pallas_call(
        body,
        out_shape=jax.ShapeDtypeStruct((2, d), jnp.float32),
        in_specs=[pl.BlockSpec(memory_space=pl.ANY),
                  pl.BlockSpec(memory_space=pl.ANY)],
        out_specs=pl.BlockSpec(memory_space=pltpu.VMEM),
        scratch_shapes=[
            pltpu.VMEM((2, _CHUNK, d), jnp.float32),
            pltpu.VMEM((2, _CHUNK, d), jnp.float32),
            pltpu.VMEM((2, d), jnp.float32),
            pltpu.VMEM((3, 2, d), jnp.float32),
            pltpu.SemaphoreType.DMA((2, 2)),
            pltpu.SemaphoreType.DMA((3,)),
            pltpu.SemaphoreType.DMA((3,)),
        ],
        compiler_params=_CompilerParams(
            collective_id=0, vmem_limit_bytes=100 * 1024 * 1024),
    )(x, dy)


# device time: 19527 ns/iter; 1.2335x vs baseline; 1.0461x over previous
import jax
import jax.numpy as jnp
from jax import lax
from jax.experimental import pallas as pl
from jax.experimental.pallas import tpu as pltpu

_DeviceIdType = getattr(pl, "DeviceIdType", None) or pltpu.DeviceIdType
_MESH = _DeviceIdType.MESH
_CompilerParams = getattr(pltpu, "CompilerParams", None) or pltpu.TPUCompilerParams

_CHUNK = 512


def kernel(x, dy, gamma):
    m, d = x.shape
    half = m // 2
    n_chunks = half // _CHUNK

    def body(x_hbm, dy_hbm, out_ref, x_buf, dy_buf, send_ref, recv_ref,
             load_sems, send_sems, recv_sems):
        my_x = lax.axis_index("x")
        my_y = lax.axis_index("y")
        peers = [(1 - my_x, my_y), (my_x, 1 - my_y), (1 - my_x, 1 - my_y)]

        barrier = pltpu.get_barrier_semaphore()
        for p in peers:
            pl.semaphore_signal(barrier, inc=1, device_id=p,
                                device_id_type=_MESH)

        row0 = my_x * half

        def start_load(h):
            slot = h % 2
            r = row0 + h * _CHUNK
            pltpu.make_async_copy(
                x_hbm.at[pl.ds(r, _CHUNK), :], x_buf.at[slot],
                load_sems.at[slot, 0]).start()
            pltpu.make_async_copy(
                dy_hbm.at[pl.ds(r, _CHUNK), :], dy_buf.at[slot],
                load_sems.at[slot, 1]).start()

        def wait_load(h):
            slot = h % 2
            r = row0 + h * _CHUNK
            pltpu.make_async_copy(
                x_hbm.at[pl.ds(r, _CHUNK), :], x_buf.at[slot],
                load_sems.at[slot, 0]).wait()
            pltpu.make_async_copy(
                dy_hbm.at[pl.ds(r, _CHUNK), :], dy_buf.at[slot],
                load_sems.at[slot, 1]).wait()

        start_load(0)
        dgamma = jnp.zeros((1, d), jnp.float32)
        dbeta = jnp.zeros((1, d), jnp.float32)
        for h in range(n_chunks):
            if h + 1 < n_chunks:
                start_load(h + 1)
            wait_load(h)
            slot = h % 2
            xv = x_buf[slot]
            dyv = dy_buf[slot]
            s1 = jnp.sum(xv, axis=1, keepdims=True)
            s2 = jnp.sum(xv * xv, axis=1, keepdims=True)
            mu = s1 * (1.0 / d)
            var = s2 * (1.0 / d) - mu * mu
            xhat = (xv - mu) * lax.rsqrt(var + 1e-5)
            dgamma = dgamma + jnp.sum(dyv * xhat, axis=0, keepdims=True)
            dbeta = dbeta + jnp.sum(dyv, axis=0, keepdims=True)
        send_ref[:, :] = jnp.concatenate([dgamma, dbeta], axis=0)

        pl.semaphore_wait(barrier, 3)

        rdmas = []
        for k, p in enumerate(peers):
            rdma = pltpu.make_async_remote_copy(
                src_ref=send_ref, dst_ref=recv_ref.at[k],
                send_sem=send_sems.at[k], recv_sem=recv_sems.at[k],
                device_id=p, device_id_type=_MESH)
            rdma.start()
            rdmas.append(rdma)
        for rdma in rdmas:
            rdma.wait()

        out_ref[:, :] = (send_ref[:, :] + recv_ref[0] + recv_ref[1]
                         + recv_ref[2])

    return pl.pallas_call(
        body,
        out_shape=jax.ShapeDtypeStruct((2, d), jnp.float32),
        in_specs=[pl.BlockSpec(memory_space=pl.ANY),
                  pl.BlockSpec(memory_space=pl.ANY)],
        out_specs=pl.BlockSpec(memory_space=pltpu.VMEM),
        scratch_shapes=[
            pltpu.VMEM((2, _CHUNK, d), jnp.float32),
            pltpu.VMEM((2, _CHUNK, d), jnp.float32),
            pltpu.VMEM((2, d), jnp.float32),
            pltpu.VMEM((3, 2, d), jnp.float32),
            pltpu.SemaphoreType.DMA((2, 2)),
            pltpu.SemaphoreType.DMA((3,)),
            pltpu.SemaphoreType.DMA((3,)),
        ],
        compiler_params=_CompilerParams(
            collective_id=0, vmem_limit_bytes=100 * 1024 * 1024),
    )(x, dy)
